# speedup vs baseline: 15.0401x; 15.0401x over previous
import functools

import jax
import jax.numpy as jnp
from jax import lax
from jax.experimental import pallas as pl
from jax.experimental.pallas import tpu as pltpu
from jax.experimental.pallas import tpu_sc as plsc

NC = 2
NS = 16
NW = NC * NS
L = 16
CHUNK = 128

_mesh = functools.partial(
    plsc.VectorSubcoreMesh,
    core_axis_name="c", subcore_axis_name="s",
    num_cores=NC, num_subcores=NS,
)


def _deg_kernel(n_pad, nchunk, interpret=False):
    @functools.partial(
        pl.kernel,
        out_type=jax.ShapeDtypeStruct((NC, n_pad), jnp.float32),
        mesh=_mesh(),
        scratch_types=[
            pltpu.VMEM_SHARED((n_pad,), jnp.float32),
            pltpu.VMEM((CHUNK,), jnp.int32),
            pltpu.VMEM((CHUNK,), jnp.float32),
        ],
        interpret=interpret,
    )
    def deg_kernel(dst_hbm, zeros1_hbm, deg_out, deg_sh, idx_v, ones_v):
        c = lax.axis_index("c")
        s = lax.axis_index("s")
        for i in range(CHUNK // L):
            ones_v[pl.ds(i * L, L)] = jnp.full((L,), 1.0, jnp.float32)

        @pl.when(s == 0)
        def _():
            pltpu.sync_copy(zeros1_hbm, deg_sh)

        plsc.subcore_barrier()

        tile = c * NS + s

        def body(j, carry):
            base = pl.multiple_of((tile * nchunk + j) * CHUNK, CHUNK)
            pltpu.sync_copy(dst_hbm.at[pl.ds(base, CHUNK)], idx_v)
            pltpu.sync_copy(ones_v, deg_sh.at[idx_v], add=True)
            return carry

        lax.fori_loop(0, nchunk, body, 0)
        plsc.subcore_barrier()

        @pl.when(s == 0)
        def _():
            pltpu.sync_copy(deg_sh, deg_out.at[c])

    return deg_kernel


def _agg_kernel(n_pad, nfeat, nchunk, interpret=False):
    rpt = n_pad // NS

    @functools.partial(
        pl.kernel,
        out_type=jax.ShapeDtypeStruct((NC, n_pad, nfeat), jnp.float32),
        mesh=_mesh(),
        scratch_types=[
            pltpu.VMEM_SHARED((n_pad, nfeat), jnp.float32),
            pltpu.VMEM((CHUNK,), jnp.int32),
            pltpu.VMEM((CHUNK,), jnp.int32),
            pltpu.VMEM((CHUNK, nfeat), jnp.float32),
            pltpu.SemaphoreType.DMA,
        ],
        interpret=interpret,
    )
    def agg_kernel(src_hbm, dst_hbm, g_hbm, zeros2_hbm, out_hbm,
                   acc_sh, sidx, didx, rows, sem):
        c = lax.axis_index("c")
        s = lax.axis_index("s")

        @pl.when(c == 0)
        def _():
            pltpu.sync_copy(g_hbm.at[pl.ds(s * rpt, rpt)],
                            acc_sh.at[pl.ds(s * rpt, rpt)])

        @pl.when(c == 1)
        def _():
            pltpu.sync_copy(zeros2_hbm.at[pl.ds(s * rpt, rpt)],
                            acc_sh.at[pl.ds(s * rpt, rpt)])

        plsc.subcore_barrier()

        tile = c * NS + s

        def body(j, carry):
            base = pl.multiple_of((tile * nchunk + j) * CHUNK, CHUNK)
            pltpu.sync_copy(src_hbm.at[pl.ds(base, CHUNK)], sidx)
            pltpu.sync_copy(dst_hbm.at[pl.ds(base, CHUNK)], didx)
            pltpu.async_copy(g_hbm.at[sidx], rows, sem).wait()
            pltpu.sync_copy(rows, acc_sh.at[didx], add=True)
            return carry

        lax.fori_loop(0, nchunk, body, 0)
        plsc.subcore_barrier()
        pltpu.sync_copy(acc_sh.at[pl.ds(s * rpt, rpt)],
                        out_hbm.at[c, pl.ds(s * rpt, rpt)])

    return agg_kernel


def _dense1(xp, w, deg3, interpret=False):
    n_pad, nfeat = xp.shape
    nhid = w.shape[1]
    nrow = n_pad // 128

    def body(x_ref, w_ref, deg_ref, g_ref, d_ref):
        h = jnp.dot(x_ref[...], w_ref[...], preferred_element_type=jnp.float32)
        d2 = lax.rsqrt(deg_ref[0] + deg_ref[1] + 1.0)
        d_ref[...] = d2
        g3 = h.reshape(nrow, 128, nhid) * d2[:, :, None]
        g_ref[...] = g3.reshape(n_pad, nhid)

    return pl.pallas_call(
        body,
        out_shape=(
            jax.ShapeDtypeStruct((n_pad, nhid), jnp.float32),
            jax.ShapeDtypeStruct((nrow, 128), jnp.float32),
        ),
        interpret=interpret,
    )(xp, w, deg3)


def _dense2(acc3, d2, b, interpret=False):
    n_pad, nhid = acc3.shape[1], acc3.shape[2]
    nrow = n_pad // 128

    def body(acc_ref, d_ref, b_ref, o_ref):
        t = (acc_ref[0] + acc_ref[1]).reshape(nrow, 128, nhid)
        o = t * d_ref[...][:, :, None] + b_ref[...]
        o_ref[...] = o.reshape(n_pad, nhid)

    return pl.pallas_call(
        body,
        out_shape=jax.ShapeDtypeStruct((n_pad, nhid), jnp.float32),
        interpret=interpret,
    )(acc3, d2, b)


def _gcn(x, edge_index, w, b, interpret=False):
    n, nfeat = x.shape
    nhid = w.shape[1]
    e = edge_index.shape[1]

    n_pad = ((n + 1 + 127) // 128) * 128
    nchunk = -(-e // (NW * CHUNK))
    ep = NW * CHUNK * nchunk

    src = jnp.concatenate(
        [edge_index[0], jnp.zeros((ep - e,), edge_index.dtype)])
    dst = jnp.concatenate(
        [edge_index[1], jnp.full((ep - e,), n, edge_index.dtype)])
    xp = jnp.pad(x, ((0, n_pad - n), (0, 0)))
    zeros1 = jnp.zeros((n_pad,), jnp.float32)
    zeros2 = jnp.zeros((n_pad, nhid), jnp.float32)

    deg2 = _deg_kernel(n_pad, nchunk, interpret)(dst, zeros1)
    g_pad, d2 = _dense1(xp, w, deg2.reshape(NC, n_pad // 128, 128), interpret)
    acc2 = _agg_kernel(n_pad, nhid, nchunk, interpret)(src, dst, g_pad, zeros2)
    out_pad = _dense2(acc2, d2, b, interpret)
    return out_pad[:n]


def kernel(x, edge_index, W, b):
    return _gcn(x, edge_index, W, b)


# speedup vs baseline: 15.0438x; 1.0002x over previous
import functools

import jax
import jax.numpy as jnp
from jax import lax
from jax.experimental import pallas as pl
from jax.experimental.pallas import tpu as pltpu
from jax.experimental.pallas import tpu_sc as plsc

NC = 2
NS = 16
NW = NC * NS
L = 16
CHUNK = 128
G = 8

_mesh = functools.partial(
    plsc.VectorSubcoreMesh,
    core_axis_name="c", subcore_axis_name="s",
    num_cores=NC, num_subcores=NS,
)


def _deg_kernel(n_pad, nchunk, interpret=False):
    @functools.partial(
        pl.kernel,
        out_type=jax.ShapeDtypeStruct((NC, n_pad), jnp.float32),
        mesh=_mesh(),
        scratch_types=[
            pltpu.VMEM_SHARED((n_pad,), jnp.float32),
            pltpu.VMEM((nchunk, CHUNK), jnp.int32),
            pltpu.VMEM((CHUNK,), jnp.float32),
            pltpu.SemaphoreType.DMA,
        ],
        interpret=interpret,
    )
    def deg_kernel(dst3_hbm, zeros1_hbm, deg_out, deg_sh, didx, ones_v, ssem):
        c = lax.axis_index("c")
        s = lax.axis_index("s")
        for i in range(CHUNK // L):
            ones_v[pl.ds(i * L, L)] = jnp.full((L,), 1.0, jnp.float32)

        @pl.when(s == 0)
        def _():
            pltpu.sync_copy(zeros1_hbm, deg_sh)

        tile = c * NS + s
        pltpu.sync_copy(dst3_hbm.at[tile], didx)
        plsc.subcore_barrier()

        def body(j, carry):
            pltpu.async_copy(ones_v, deg_sh.at[didx.at[j]], ssem, add=True)
            return carry

        lax.fori_loop(0, nchunk, body, 0)

        def drain(j, carry):
            pltpu.make_async_copy(ones_v, deg_sh.at[didx.at[0]], ssem).wait()
            return carry

        lax.fori_loop(0, nchunk, drain, 0)
        plsc.subcore_barrier()

        @pl.when(s == 0)
        def _():
            pltpu.sync_copy(deg_sh, deg_out.at[c])

    return deg_kernel


def _agg_kernel(n_pad, nfeat, nchunk, interpret=False):
    rpt = n_pad // NS

    ngrp = nchunk // G

    @functools.partial(
        pl.kernel,
        out_type=jax.ShapeDtypeStruct((NC, n_pad, nfeat), jnp.float32),
        mesh=_mesh(),
        scratch_types=[
            pltpu.VMEM_SHARED((n_pad, nfeat), jnp.float32),
            pltpu.VMEM((G, CHUNK), jnp.int32),
            pltpu.VMEM((G, CHUNK), jnp.int32),
            pltpu.VMEM((2, CHUNK, nfeat), jnp.float32),
            pltpu.SemaphoreType.DMA,
            pltpu.SemaphoreType.DMA,
        ],
        interpret=interpret,
    )
    def agg_kernel(src3_hbm, dst3_hbm, g_hbm, zeros2_hbm, out_hbm,
                   acc_sh, sidx, didx, rows, gsem0, gsem1):
        c = lax.axis_index("c")
        s = lax.axis_index("s")
        gsem = (gsem0, gsem1)

        @pl.when(c == 0)
        def _():
            pltpu.sync_copy(g_hbm.at[pl.ds(s * rpt, rpt)],
                            acc_sh.at[pl.ds(s * rpt, rpt)])

        @pl.when(c == 1)
        def _():
            pltpu.sync_copy(zeros2_hbm.at[pl.ds(s * rpt, rpt)],
                            acc_sh.at[pl.ds(s * rpt, rpt)])

        tile = c * NS + s
        plsc.subcore_barrier()

        def group(k, carry):
            pltpu.sync_copy(src3_hbm.at[tile, pl.ds(k * G, G)], sidx)
            pltpu.sync_copy(dst3_hbm.at[tile, pl.ds(k * G, G)], didx)
            pltpu.async_copy(g_hbm.at[sidx.at[0]], rows.at[0], gsem0)
            for jj in range(G):
                b = jj % 2
                if jj < G - 1:
                    pltpu.async_copy(g_hbm.at[sidx.at[jj + 1]],
                                     rows.at[1 - b], gsem[1 - b])
                pltpu.make_async_copy(g_hbm.at[sidx.at[jj]], rows.at[b],
                                      gsem[b]).wait()
                pltpu.sync_copy(rows.at[b], acc_sh.at[didx.at[jj]], add=True)
            return carry

        lax.fori_loop(0, ngrp, group, 0)
        plsc.subcore_barrier()
        pltpu.sync_copy(acc_sh.at[pl.ds(s * rpt, rpt)],
                        out_hbm.at[c, pl.ds(s * rpt, rpt)])

    return agg_kernel


def _dense1(xp, w, deg3, interpret=False):
    n_pad, nfeat = xp.shape
    nhid = w.shape[1]
    nrow = n_pad // 128

    def body(x_ref, w_ref, deg_ref, g_ref, d_ref):
        h = jnp.dot(x_ref[...], w_ref[...], preferred_element_type=jnp.float32)
        d2 = lax.rsqrt(deg_ref[0] + deg_ref[1] + 1.0)
        d_ref[...] = d2
        g3 = h.reshape(nrow, 128, nhid) * d2[:, :, None]
        g_ref[...] = g3.reshape(n_pad, nhid)

    return pl.pallas_call(
        body,
        out_shape=(
            jax.ShapeDtypeStruct((n_pad, nhid), jnp.float32),
            jax.ShapeDtypeStruct((nrow, 128), jnp.float32),
        ),
        interpret=interpret,
    )(xp, w, deg3)


def _dense2(acc3, d2, b, interpret=False):
    n_pad, nhid = acc3.shape[1], acc3.shape[2]
    nrow = n_pad // 128

    def body(acc_ref, d_ref, b_ref, o_ref):
        t = (acc_ref[0] + acc_ref[1]).reshape(nrow, 128, nhid)
        o = t * d_ref[...][:, :, None] + b_ref[...]
        o_ref[...] = o.reshape(n_pad, nhid)

    return pl.pallas_call(
        body,
        out_shape=jax.ShapeDtypeStruct((n_pad, nhid), jnp.float32),
        interpret=interpret,
    )(acc3, d2, b)


def _gcn(x, edge_index, w, b, interpret=False):
    n, nfeat = x.shape
    nhid = w.shape[1]
    e = edge_index.shape[1]

    n_pad = ((n + 1 + 127) // 128) * 128
    nchunk = -(-e // (NW * CHUNK))
    nchunk = -(-nchunk // G) * G
    ep = NW * CHUNK * nchunk

    src3 = jnp.concatenate(
        [edge_index[0], jnp.zeros((ep - e,), edge_index.dtype)]
    ).reshape(NW, nchunk, CHUNK)
    dst3 = jnp.concatenate(
        [edge_index[1], jnp.full((ep - e,), n, edge_index.dtype)]
    ).reshape(NW, nchunk, CHUNK)
    xp = jnp.pad(x, ((0, n_pad - n), (0, 0)))
    zeros1 = jnp.zeros((n_pad,), jnp.float32)
    zeros2 = jnp.zeros((n_pad, nhid), jnp.float32)

    deg2 = _deg_kernel(n_pad, nchunk, interpret)(dst3, zeros1)
    g_pad, d2 = _dense1(xp, w, deg2.reshape(NC, n_pad // 128, 128), interpret)
    acc2 = _agg_kernel(n_pad, nhid, nchunk, interpret)(src3, dst3, g_pad, zeros2)
    out_pad = _dense2(acc2, d2, b, interpret)
    return out_pad[:n]


def kernel(x, edge_index, W, b):
    return _gcn(x, edge_index, W, b)


# speedup vs baseline: 15.2651x; 1.0147x over previous
import functools

import jax
import jax.numpy as jnp
from jax import lax
from jax.experimental import pallas as pl
from jax.experimental.pallas import tpu as pltpu
from jax.experimental.pallas import tpu_sc as plsc

NC = 2
NS = 16
NW = NC * NS
L = 16
CHUNK = 128
G = 8

_mesh = functools.partial(
    plsc.VectorSubcoreMesh,
    core_axis_name="c", subcore_axis_name="s",
    num_cores=NC, num_subcores=NS,
)


def _deg_kernel(n_pad, nchunk, interpret=False):
    @functools.partial(
        pl.kernel,
        out_type=jax.ShapeDtypeStruct((NC, n_pad), jnp.float32),
        mesh=_mesh(),
        scratch_types=[
            pltpu.VMEM_SHARED((n_pad,), jnp.float32),
            pltpu.VMEM((nchunk, CHUNK), jnp.int32),
            pltpu.VMEM((CHUNK,), jnp.float32),
            pltpu.SemaphoreType.DMA,
        ],
        interpret=interpret,
    )
    def deg_kernel(dst3_hbm, zeros1_hbm, deg_out, deg_sh, didx, ones_v, ssem):
        c = lax.axis_index("c")
        s = lax.axis_index("s")
        for i in range(CHUNK // L):
            ones_v[pl.ds(i * L, L)] = jnp.full((L,), 1.0, jnp.float32)

        @pl.when(s == 0)
        def _():
            pltpu.sync_copy(zeros1_hbm, deg_sh)

        tile = c * NS + s
        pltpu.sync_copy(dst3_hbm.at[tile], didx)
        plsc.subcore_barrier()

        def body(j, carry):
            pltpu.async_copy(ones_v, deg_sh.at[didx.at[j]], ssem, add=True)
            return carry

        lax.fori_loop(0, nchunk, body, 0)

        def drain(j, carry):
            pltpu.make_async_copy(ones_v, deg_sh.at[didx.at[0]], ssem).wait()
            return carry

        lax.fori_loop(0, nchunk, drain, 0)
        plsc.subcore_barrier()

        @pl.when(s == 0)
        def _():
            pltpu.sync_copy(deg_sh, deg_out.at[c])

    return deg_kernel


def _agg_kernel(n_pad, nfeat, nchunk, interpret=False):
    rpt = n_pad // NS

    ngrp = nchunk // G

    @functools.partial(
        pl.kernel,
        out_type=jax.ShapeDtypeStruct((NC, n_pad, nfeat), jnp.float32),
        mesh=_mesh(),
        scratch_types=[
            pltpu.VMEM_SHARED((n_pad, nfeat), jnp.float32),
            pltpu.VMEM((G, CHUNK), jnp.int32),
            pltpu.VMEM((G, CHUNK), jnp.int32),
            pltpu.VMEM((2, CHUNK, nfeat), jnp.float32),
            pltpu.SemaphoreType.DMA,
            pltpu.SemaphoreType.DMA,
        ],
        interpret=interpret,
    )
    def agg_kernel(src3_hbm, dst3_hbm, g_hbm, zeros2_hbm, out_hbm,
                   acc_sh, sidx, didx, rows, gsem0, gsem1):
        c = lax.axis_index("c")
        s = lax.axis_index("s")
        gsem = (gsem0, gsem1)

        @pl.when(c == 0)
        def _():
            pltpu.sync_copy(g_hbm.at[pl.ds(s * rpt, rpt)],
                            acc_sh.at[pl.ds(s * rpt, rpt)])

        @pl.when(c == 1)
        def _():
            pltpu.sync_copy(zeros2_hbm.at[pl.ds(s * rpt, rpt)],
                            acc_sh.at[pl.ds(s * rpt, rpt)])

        tile = c * NS + s
        plsc.subcore_barrier()

        def group(k, carry):
            pltpu.sync_copy(src3_hbm.at[tile, pl.ds(k * G, G)], sidx)
            pltpu.sync_copy(dst3_hbm.at[tile, pl.ds(k * G, G)], didx)
            pltpu.async_copy(g_hbm.at[sidx.at[0]], rows.at[0], gsem0)
            for jj in range(G):
                b = jj % 2
                if jj < G - 1:
                    pltpu.async_copy(g_hbm.at[sidx.at[jj + 1]],
                                     rows.at[1 - b], gsem[1 - b])
                pltpu.make_async_copy(g_hbm.at[sidx.at[jj]], rows.at[b],
                                      gsem[b]).wait()
                pltpu.sync_copy(rows.at[b], acc_sh.at[didx.at[jj]], add=True)
            return carry

        lax.fori_loop(0, ngrp, group, 0)
        plsc.subcore_barrier()
        pltpu.sync_copy(acc_sh.at[pl.ds(s * rpt, rpt)],
                        out_hbm.at[c, pl.ds(s * rpt, rpt)])

    return agg_kernel


def _dense1(xp, w, deg3, interpret=False):
    n_pad, nfeat = xp.shape
    nhid = w.shape[1]
    nrow = n_pad // 128

    def body(x_ref, w_ref, deg_ref, g_ref, d_ref):
        h = jnp.dot(x_ref[...], w_ref[...], preferred_element_type=jnp.float32)
        d2 = lax.rsqrt(deg_ref[0] + deg_ref[1] + 1.0)
        d_ref[...] = d2
        g3 = h.reshape(nrow, 128, nhid) * d2[:, :, None]
        g_ref[...] = g3.reshape(n_pad, nhid)

    return pl.pallas_call(
        body,
        out_shape=(
            jax.ShapeDtypeStruct((n_pad, nhid), jnp.float32),
            jax.ShapeDtypeStruct((nrow, 128), jnp.float32),
        ),
        interpret=interpret,
    )(xp, w, deg3)


def _dense2(acc3, d2, b, interpret=False):
    n_pad, nhid = acc3.shape[1], acc3.shape[2]
    nrow = n_pad // 128

    def body(acc_ref, d_ref, b_ref, o_ref):
        t = (acc_ref[0] + acc_ref[1]).reshape(nrow, 128, nhid)
        o = t * d_ref[...][:, :, None] + b_ref[...]
        o_ref[...] = o.reshape(n_pad, nhid)

    return pl.pallas_call(
        body,
        out_shape=jax.ShapeDtypeStruct((n_pad, nhid), jnp.float32),
        interpret=interpret,
    )(acc3, d2, b)


def _gcn(x, edge_index, w, b, interpret=False):
    n, nfeat = x.shape
    nhid = w.shape[1]
    e = edge_index.shape[1]

    n_pad = ((n + 1 + 127) // 128) * 128
    nchunk = -(-e // (NW * CHUNK))
    nchunk = -(-nchunk // G) * G
    ep = NW * CHUNK * nchunk

    src3 = jnp.concatenate(
        [edge_index[0], jnp.zeros((ep - e,), edge_index.dtype)]
    ).reshape(NW, nchunk, CHUNK)
    pad_dst = (n + jnp.arange(ep - e, dtype=edge_index.dtype)
               % jnp.int32(n_pad - n))
    dst3 = jnp.concatenate([edge_index[1], pad_dst]).reshape(NW, nchunk, CHUNK)
    xp = jnp.pad(x, ((0, n_pad - n), (0, 0)))
    zeros1 = jnp.zeros((n_pad,), jnp.float32)
    zeros2 = jnp.zeros((n_pad, nhid), jnp.float32)

    deg2 = _deg_kernel(n_pad, nchunk, interpret)(dst3, zeros1)
    g_pad, d2 = _dense1(xp, w, deg2.reshape(NC, n_pad // 128, 128), interpret)
    acc2 = _agg_kernel(n_pad, nhid, nchunk, interpret)(src3, dst3, g_pad, zeros2)
    out_pad = _dense2(acc2, d2, b, interpret)
    return out_pad[:n]


def kernel(x, edge_index, W, b):
    return _gcn(x, edge_index, W, b)


# speedup vs baseline: 29.7248x; 1.9472x over previous
import functools

import jax
import jax.numpy as jnp
from jax import lax
from jax.experimental import pallas as pl
from jax.experimental.pallas import tpu as pltpu
from jax.experimental.pallas import tpu_sc as plsc

NC = 2
NS = 16
NW = NC * NS
L = 16
CHUNK = 128
G = 8

_mesh = functools.partial(
    plsc.VectorSubcoreMesh,
    core_axis_name="c", subcore_axis_name="s",
    num_cores=NC, num_subcores=NS,
)


def _deg_kernel(n_pad, nchunk, interpret=False):
    nch2 = nchunk // NC

    @functools.partial(
        pl.kernel,
        out_type=jax.ShapeDtypeStruct((NC, n_pad), jnp.float32),
        mesh=_mesh(),
        scratch_types=[
            pltpu.VMEM_SHARED((n_pad,), jnp.float32),
            pltpu.VMEM((nch2, CHUNK), jnp.int32),
            pltpu.VMEM((CHUNK,), jnp.float32),
            pltpu.SemaphoreType.DMA,
        ],
        interpret=interpret,
    )
    def deg_kernel(dst3_hbm, zeros1_hbm, deg_out, deg_sh, didx, ones_v, ssem):
        c = lax.axis_index("c")
        s = lax.axis_index("s")
        for i in range(CHUNK // L):
            ones_v[pl.ds(i * L, L)] = jnp.full((L,), 1.0, jnp.float32)

        @pl.when(s == 0)
        def _():
            pltpu.sync_copy(zeros1_hbm, deg_sh)

        pltpu.sync_copy(dst3_hbm.at[s, pl.ds(c * nch2, nch2)], didx)
        plsc.subcore_barrier()

        def body(j, carry):
            pltpu.async_copy(ones_v, deg_sh.at[didx.at[j]], ssem, add=True)
            return carry

        lax.fori_loop(0, nch2, body, 0)

        def drain(j, carry):
            pltpu.make_async_copy(ones_v, deg_sh.at[didx.at[0]], ssem).wait()
            return carry

        lax.fori_loop(0, nch2, drain, 0)
        plsc.subcore_barrier()

        @pl.when(s == 0)
        def _():
            pltpu.sync_copy(deg_sh, deg_out.at[c])

    return deg_kernel


def _agg_kernel(n_pad, nfeat, nchunk, interpret=False):
    hw = nfeat // NC
    rpt = n_pad // NS
    ngrp = nchunk // G

    @functools.partial(
        pl.kernel,
        out_type=jax.ShapeDtypeStruct((NC, n_pad, hw), jnp.float32),
        mesh=_mesh(),
        scratch_types=[
            pltpu.VMEM_SHARED((n_pad, hw), jnp.float32),
            pltpu.VMEM_SHARED((n_pad, hw), jnp.float32),
            pltpu.VMEM((G, CHUNK), jnp.int32),
            pltpu.VMEM((G, CHUNK), jnp.int32),
            pltpu.VMEM((2, CHUNK, hw), jnp.float32),
            pltpu.SemaphoreType.DMA,
            pltpu.SemaphoreType.DMA,
        ],
        interpret=interpret,
    )
    def agg_kernel(src3_hbm, dst3_hbm, g0_hbm, g1_hbm, out_hbm,
                   g_sh, acc_sh, sidx, didx, rows, gsem0, gsem1):
        c = lax.axis_index("c")
        s = lax.axis_index("s")
        gsem = (gsem0, gsem1)

        @pl.when(c == 0)
        def _():
            pltpu.sync_copy(g0_hbm.at[pl.ds(s * rpt, rpt)],
                            g_sh.at[pl.ds(s * rpt, rpt)])
            pltpu.sync_copy(g0_hbm.at[pl.ds(s * rpt, rpt)],
                            acc_sh.at[pl.ds(s * rpt, rpt)])

        @pl.when(c == 1)
        def _():
            pltpu.sync_copy(g1_hbm.at[pl.ds(s * rpt, rpt)],
                            g_sh.at[pl.ds(s * rpt, rpt)])
            pltpu.sync_copy(g1_hbm.at[pl.ds(s * rpt, rpt)],
                            acc_sh.at[pl.ds(s * rpt, rpt)])

        plsc.subcore_barrier()

        def group(k, carry):
            pltpu.sync_copy(src3_hbm.at[s, pl.ds(k * G, G)], sidx)
            pltpu.sync_copy(dst3_hbm.at[s, pl.ds(k * G, G)], didx)
            pltpu.async_copy(g_sh.at[sidx.at[0]], rows.at[0], gsem0)
            for jj in range(G):
                b = jj % 2
                if jj < G - 1:
                    pltpu.async_copy(g_sh.at[sidx.at[jj + 1]],
                                     rows.at[1 - b], gsem[1 - b])
                pltpu.make_async_copy(g_sh.at[sidx.at[jj]], rows.at[b],
                                      gsem[b]).wait()
                pltpu.sync_copy(rows.at[b], acc_sh.at[didx.at[jj]], add=True)
            return carry

        lax.fori_loop(0, ngrp, group, 0)
        plsc.subcore_barrier()
        pltpu.sync_copy(acc_sh.at[pl.ds(s * rpt, rpt)],
                        out_hbm.at[c, pl.ds(s * rpt, rpt)])

    return agg_kernel


def _dense1(xp, w, deg3, interpret=False):
    n_pad, nfeat = xp.shape
    nhid = w.shape[1]
    hw = nhid // NC
    nrow = n_pad // 128

    def body(x_ref, w_ref, deg_ref, g0_ref, g1_ref, d_ref):
        h = jnp.dot(x_ref[...], w_ref[...], preferred_element_type=jnp.float32)
        d2 = lax.rsqrt(deg_ref[0] + deg_ref[1] + 1.0)
        d_ref[...] = d2
        g3 = h.reshape(nrow, 128, nhid) * d2[:, :, None]
        g = g3.reshape(n_pad, nhid)
        g0_ref[...] = g[:, :hw]
        g1_ref[...] = g[:, hw:]

    return pl.pallas_call(
        body,
        out_shape=(
            jax.ShapeDtypeStruct((n_pad, hw), jnp.float32),
            jax.ShapeDtypeStruct((n_pad, hw), jnp.float32),
            jax.ShapeDtypeStruct((nrow, 128), jnp.float32),
        ),
        interpret=interpret,
    )(xp, w, deg3)


def _dense2(acc3, d2, b, interpret=False):
    n_pad, nhid = acc3.shape[1], acc3.shape[0] * acc3.shape[2]
    nrow = n_pad // 128

    def body(acc_ref, d_ref, b_ref, o_ref):
        t = jnp.concatenate([acc_ref[0], acc_ref[1]], axis=-1)
        t = t.reshape(nrow, 128, nhid)
        o = t * d_ref[...][:, :, None] + b_ref[...]
        o_ref[...] = o.reshape(n_pad, nhid)

    return pl.pallas_call(
        body,
        out_shape=jax.ShapeDtypeStruct((n_pad, nhid), jnp.float32),
        interpret=interpret,
    )(acc3, d2, b)


def _gcn(x, edge_index, w, b, interpret=False):
    n, nfeat = x.shape
    nhid = w.shape[1]
    e = edge_index.shape[1]

    n_pad = ((n + 1 + 127) // 128) * 128
    nchunk = -(-e // (NS * CHUNK))
    nchunk = -(-nchunk // (NC * G)) * (NC * G)
    ep = NS * CHUNK * nchunk

    src3 = jnp.concatenate(
        [edge_index[0], jnp.zeros((ep - e,), edge_index.dtype)]
    ).reshape(NS, nchunk, CHUNK)
    pad_dst = (n + jnp.arange(ep - e, dtype=edge_index.dtype)
               % jnp.int32(n_pad - n))
    dst3 = jnp.concatenate([edge_index[1], pad_dst]).reshape(NS, nchunk, CHUNK)
    xp = jnp.pad(x, ((0, n_pad - n), (0, 0)))
    zeros1 = jnp.zeros((n_pad,), jnp.float32)

    deg2 = _deg_kernel(n_pad, nchunk, interpret)(dst3, zeros1)
    g0, g1, d2 = _dense1(xp, w, deg2.reshape(NC, n_pad // 128, 128), interpret)
    acc2 = _agg_kernel(n_pad, nhid, nchunk, interpret)(src3, dst3, g0, g1)
    out_pad = _dense2(acc2, d2, b, interpret)
    return out_pad[:n]


def kernel(x, edge_index, W, b):
    return _gcn(x, edge_index, W, b)
